# baseline (speedup 1.0000x reference)
import functools

import jax
import jax.numpy as jnp
from jax import lax
from jax.experimental import pallas as pl
from jax.experimental.pallas import tpu as pltpu
from jax.experimental.pallas import tpu_sc as plsc

B, S, H = 4, 2048, 768
N = B * S
EPS = 1e-12

NC, NS, L = 2, 16, 16
NW = NC * NS
TOK_PER_W = N // NW
CHUNK = 32
NCHUNK = TOK_PER_W // CHUNK
HV = H // L


def _rsqrt(x):
    xh = 0.5 * x
    i = lax.bitcast_convert_type(x, jnp.int32)
    i = jnp.int32(0x5F3759DF) - (i >> 1)
    y = lax.bitcast_convert_type(i, jnp.float32)
    y = y * (1.5 - xh * y * y)
    y = y * (1.5 - xh * y * y)
    y = y * (1.5 - xh * y * y)
    return y


def _body(ids_w, ids_t, ids_p, ids_g, ids_k,
          word_hbm, type_hbm, pos_hbm, seg_hbm, spk_hbm,
          gamma_hbm, beta_hbm, out_hbm,
          iw, it, ip, ig, ik,
          bw, bt, bp, bg, bk,
          gbuf, bbuf, sem):
    wid = lax.axis_index("s") * NC + lax.axis_index("c")
    base = wid * TOK_PER_W

    pltpu.sync_copy(gamma_hbm, gbuf)
    pltpu.sync_copy(beta_hbm, bbuf)

    def chunk_body(c, carry):
        cbase = base + c * CHUNK
        sl = pl.ds(cbase, CHUNK)
        pltpu.sync_copy(ids_w.at[sl], iw)
        pltpu.sync_copy(ids_t.at[sl], it)
        pltpu.sync_copy(ids_p.at[sl], ip)
        pltpu.sync_copy(ids_g.at[sl], ig)
        pltpu.sync_copy(ids_k.at[sl], ik)

        d1 = pltpu.async_copy(word_hbm.at[iw], bw, sem)
        d2 = pltpu.async_copy(type_hbm.at[it], bt, sem)
        d3 = pltpu.async_copy(pos_hbm.at[ip], bp, sem)
        d4 = pltpu.async_copy(seg_hbm.at[ig], bg, sem)
        d5 = pltpu.async_copy(spk_hbm.at[ik], bk, sem)
        d1.wait()
        d2.wait()
        d3.wait()
        d4.wait()
        d5.wait()

        def row_body(r, carry2):
            s = jnp.zeros((L,), jnp.float32)
            ss = jnp.zeros((L,), jnp.float32)
            for j in range(HV):
                hs = pl.ds(j * L, L)
                v = (bw[r, hs] + bt[r, hs] + bp[r, hs]
                     + bg[r, hs] + bk[r, hs])
                bw[r, hs] = v
                s = s + v
                ss = ss + v * v
            mean = lax.reduce_sum_p.bind(s, axes=(0,)) * (1.0 / H)
            msq = lax.reduce_sum_p.bind(ss, axes=(0,)) * (1.0 / H)
            rstd = _rsqrt(msq - mean * mean + EPS)
            for j in range(HV):
                hs = pl.ds(j * L, L)
                bw[r, hs] = (bw[r, hs] - mean) * rstd * gbuf[hs] + bbuf[hs]
            return carry2

        lax.fori_loop(0, CHUNK, row_body, 0)
        pltpu.sync_copy(bw, out_hbm.at[sl])
        return carry

    lax.fori_loop(0, NCHUNK, chunk_body, 0)


@jax.jit
def _run(ids_w, ids_t, ids_p, ids_g, ids_k,
         word_emb, type_emb, pos_emb, seg_emb, spk_emb, ln_gamma, ln_beta):
    mesh = plsc.VectorSubcoreMesh(core_axis_name="c", subcore_axis_name="s",
                                  num_cores=NC, num_subcores=NS)
    f = pl.kernel(
        _body,
        out_type=jax.ShapeDtypeStruct((N, H), jnp.float32),
        mesh=mesh,
        scratch_types=[
            pltpu.VMEM((CHUNK,), jnp.int32),
            pltpu.VMEM((CHUNK,), jnp.int32),
            pltpu.VMEM((CHUNK,), jnp.int32),
            pltpu.VMEM((CHUNK,), jnp.int32),
            pltpu.VMEM((CHUNK,), jnp.int32),
            pltpu.VMEM((CHUNK, H), jnp.float32),
            pltpu.VMEM((CHUNK, H), jnp.float32),
            pltpu.VMEM((CHUNK, H), jnp.float32),
            pltpu.VMEM((CHUNK, H), jnp.float32),
            pltpu.VMEM((CHUNK, H), jnp.float32),
            pltpu.VMEM((H,), jnp.float32),
            pltpu.VMEM((H,), jnp.float32),
            pltpu.SemaphoreType.DMA,
        ],
        compiler_params=pltpu.CompilerParams(needs_layout_passes=False),
        name="wsw_embed_ln",
    )
    return f(ids_w, ids_t, ids_p, ids_g, ids_k,
             word_emb, type_emb, pos_emb, seg_emb, spk_emb,
             ln_gamma, ln_beta)


def kernel(input_ids, token_type_ids, position_ids, segment_ids, speaker_ids,
           word_emb, type_emb, pos_emb, seg_emb, spk_emb, ln_gamma, ln_beta):
    out = _run(
        input_ids.reshape(-1).astype(jnp.int32),
        token_type_ids.reshape(-1).astype(jnp.int32),
        position_ids.reshape(-1).astype(jnp.int32),
        segment_ids.reshape(-1).astype(jnp.int32),
        speaker_ids.reshape(-1).astype(jnp.int32),
        word_emb, type_emb, pos_emb, seg_emb, spk_emb, ln_gamma, ln_beta)
    return out.reshape(B, S, H)


# speedup vs baseline: 1.6245x; 1.6245x over previous
import jax
import jax.numpy as jnp
from jax import lax
from jax.experimental import pallas as pl
from jax.experimental.pallas import tpu as pltpu
from jax.experimental.pallas import tpu_sc as plsc

B, S, H = 4, 2048, 768
N = B * S
EPS = 1e-12

NC, NS, L = 2, 16, 16
NW = NC * NS
TOK_PER_W = N // NW
CHUNK = 8
NCHUNK = TOK_PER_W // CHUNK
NPAIR = NCHUNK // 2
HV = H // L
TYPES, MAXSEG, MAXSPK = 2, 64, 16


def _rsqrt(x):
    xh = 0.5 * x
    i = lax.bitcast_convert_type(x, jnp.int32)
    i = jnp.int32(0x5F3759DF) - (i >> 1)
    y = lax.bitcast_convert_type(i, jnp.float32)
    y = y * (1.5 - xh * y * y)
    y = y * (1.5 - xh * y * y)
    y = y * (1.5 - xh * y * y)
    return y


def _body(ids_w, ids_t, ids_p, ids_g, ids_k,
          word_hbm, type_hbm, pos_hbm, seg_hbm, spk_hbm,
          gamma_hbm, beta_hbm, out_hbm,
          iw, it, ip, ig, ik,
          tb, gb, kb, gbuf, bbuf,
          bw0, bp0, bw1, bp1, ob0, ob1,
          semg0, semg1, semo0, semo1, sems):
    wid = lax.axis_index("s") * NC + lax.axis_index("c")
    rbase = wid * NCHUNK

    rsl = pl.ds(rbase, NCHUNK)
    for src, dst in ((type_hbm, tb), (seg_hbm, gb), (spk_hbm, kb),
                     (gamma_hbm, gbuf), (beta_hbm, bbuf),
                     (ids_w.at[rsl], iw), (ids_t.at[rsl], it),
                     (ids_p.at[rsl], ip), (ids_g.at[rsl], ig),
                     (ids_k.at[rsl], ik)):
        pltpu.async_copy(src, dst, sems)
    for src, dst in ((type_hbm, tb), (seg_hbm, gb), (spk_hbm, kb),
                     (gamma_hbm, gbuf), (beta_hbm, bbuf),
                     (ids_w.at[rsl], iw), (ids_t.at[rsl], it),
                     (ids_p.at[rsl], ip), (ids_g.at[rsl], ig),
                     (ids_k.at[rsl], ik)):
        pltpu.make_async_copy(src, dst, sems).wait()

    pltpu.async_copy(word_hbm.at[iw.at[0]], bw0, semg0)
    pltpu.async_copy(pos_hbm.at[ip.at[0]], bp0, semg0)
    pltpu.async_copy(word_hbm.at[iw.at[1]], bw1, semg1)
    pltpu.async_copy(pos_hbm.at[ip.at[1]], bp1, semg1)

    lanes = lax.iota(jnp.int32, L)

    def compute_chunk(c, bw, bp, ob):
        def row_body(r, carry):
            cf = jnp.full((L,), c, jnp.int32)
            rf = jnp.full((L,), r, jnp.int32)
            tidv = plsc.load_gather(it, [cf, rf])
            gidv = plsc.load_gather(ig, [cf, rf])
            kidv = plsc.load_gather(ik, [cf, rf])
            s = jnp.zeros((L,), jnp.float32)
            ss = jnp.zeros((L,), jnp.float32)
            for j in range(HV):
                hs = pl.ds(j * L, L)
                col = j * L + lanes
                v = (bw[r, hs] + bp[r, hs]
                     + plsc.load_gather(tb, [tidv, col])
                     + plsc.load_gather(gb, [gidv, col])
                     + plsc.load_gather(kb, [kidv, col]))
                ob[r, hs] = v
                s = s + v
                ss = ss + v * v
            mean = lax.reduce_sum_p.bind(s, axes=(0,)) * (1.0 / H)
            msq = lax.reduce_sum_p.bind(ss, axes=(0,)) * (1.0 / H)
            rstd = _rsqrt(msq - mean * mean + EPS)
            for j in range(HV):
                hs = pl.ds(j * L, L)
                ob[r, hs] = (ob[r, hs] - mean) * rstd * gbuf[hs] + bbuf[hs]
            return carry
        lax.fori_loop(0, CHUNK, row_body, 0)

    def pair_body(i, carry):
        for c, bw, bp, ob, semg, semo in (
                (2 * i, bw0, bp0, ob0, semg0, semo0),
                (2 * i + 1, bw1, bp1, ob1, semg1, semo1)):
            osl = pl.ds((rbase + c) * CHUNK, CHUNK)
            pltpu.make_async_copy(word_hbm.at[iw.at[c]], bw, semg).wait()
            pltpu.make_async_copy(pos_hbm.at[ip.at[c]], bp, semg).wait()

            @pl.when(i > 0)
            def _():
                pltpu.make_async_copy(ob, out_hbm.at[osl], semo).wait()

            compute_chunk(c, bw, bp, ob)
            pltpu.async_copy(ob, out_hbm.at[osl], semo)

            @pl.when(i < NPAIR - 1)
            def _():
                pltpu.async_copy(word_hbm.at[iw.at[c + 2]], bw, semg)
                pltpu.async_copy(pos_hbm.at[ip.at[c + 2]], bp, semg)
        return carry

    lax.fori_loop(0, NPAIR, pair_body, 0)

    pltpu.make_async_copy(
        ob0, out_hbm.at[pl.ds((rbase + NCHUNK - 2) * CHUNK, CHUNK)],
        semo0).wait()
    pltpu.make_async_copy(
        ob1, out_hbm.at[pl.ds((rbase + NCHUNK - 1) * CHUNK, CHUNK)],
        semo1).wait()


@jax.jit
def _run(ids_w, ids_t, ids_p, ids_g, ids_k,
         word_emb, type_emb, pos_emb, seg_emb, spk_emb, ln_gamma, ln_beta):
    mesh = plsc.VectorSubcoreMesh(core_axis_name="c", subcore_axis_name="s",
                                  num_cores=NC, num_subcores=NS)
    f = pl.kernel(
        _body,
        out_type=jax.ShapeDtypeStruct((N, H), jnp.float32),
        mesh=mesh,
        scratch_types=[
            pltpu.VMEM((NCHUNK, CHUNK), jnp.int32),
            pltpu.VMEM((NCHUNK, CHUNK), jnp.int32),
            pltpu.VMEM((NCHUNK, CHUNK), jnp.int32),
            pltpu.VMEM((NCHUNK, CHUNK), jnp.int32),
            pltpu.VMEM((NCHUNK, CHUNK), jnp.int32),
            pltpu.VMEM((TYPES, H), jnp.float32),
            pltpu.VMEM((MAXSEG, H), jnp.float32),
            pltpu.VMEM((MAXSPK, H), jnp.float32),
            pltpu.VMEM((H,), jnp.float32),
            pltpu.VMEM((H,), jnp.float32),
            pltpu.VMEM((CHUNK, H), jnp.float32),
            pltpu.VMEM((CHUNK, H), jnp.float32),
            pltpu.VMEM((CHUNK, H), jnp.float32),
            pltpu.VMEM((CHUNK, H), jnp.float32),
            pltpu.VMEM((CHUNK, H), jnp.float32),
            pltpu.VMEM((CHUNK, H), jnp.float32),
            pltpu.SemaphoreType.DMA,
            pltpu.SemaphoreType.DMA,
            pltpu.SemaphoreType.DMA,
            pltpu.SemaphoreType.DMA,
            pltpu.SemaphoreType.DMA,
        ],
        compiler_params=pltpu.CompilerParams(needs_layout_passes=False),
        name="wsw_embed_ln",
    )
    return f(ids_w, ids_t, ids_p, ids_g, ids_k,
             word_emb, type_emb, pos_emb, seg_emb, spk_emb,
             ln_gamma, ln_beta)


def kernel(input_ids, token_type_ids, position_ids, segment_ids, speaker_ids,
           word_emb, type_emb, pos_emb, seg_emb, spk_emb, ln_gamma, ln_beta):
    def prep(x):
        return x.reshape(N // CHUNK, CHUNK).astype(jnp.int32)
    out = _run(
        prep(input_ids), prep(token_type_ids), prep(position_ids),
        prep(segment_ids), prep(speaker_ids),
        word_emb, type_emb, pos_emb, seg_emb, spk_emb, ln_gamma, ln_beta)
    return out.reshape(B, S, H)
